# Initial kernel scaffold; baseline (speedup 1.0000x reference)
import jax
import jax.numpy as jnp
from jax.experimental import pallas as pl


def kernel(z, books):
    raise NotImplementedError("write your pallas kernel here")


# speedup vs baseline: 1.1516x; 1.1516x over previous
import functools

import jax
import jax.numpy as jnp
from jax import lax
from jax.experimental import pallas as pl
from jax.experimental.pallas import tpu as pltpu
from jax.experimental.pallas import tpu_sc as plsc

_B, _D, _T = 16, 256, 576
_M = _B * _T
_K = 8192
_NBOOKS = 8

_RT = 1152
_KT = 1024
_NR = _M // _RT
_NK = _K // _KT

_NC, _NS = 2, 16
_NW = _NC * _NS
_BPW = _M // _NW


def _vq_tile(r, emb_ref, idx_ref, best_ref, bidx_ref):
    k = pl.program_id(1)
    emb = emb_ref[...]
    logits = lax.dot_general(
        r.astype(jnp.bfloat16), emb.astype(jnp.bfloat16),
        (((1,), (1,)), ((), ())),
        preferred_element_type=jnp.float32)
    nrm = 0.5 * jnp.sum(emb * emb, axis=1)
    logits = logits - nrm[None, :]
    tmax = jnp.max(logits, axis=1, keepdims=True)
    col = lax.broadcasted_iota(jnp.int32, logits.shape, 1)
    targ = jnp.min(jnp.where(logits == tmax, col + k * _KT, _K),
                   axis=1, keepdims=True)
    better = tmax > best_ref[...]
    tmax_b16 = tmax.astype(jnp.bfloat16).astype(jnp.float32)
    best_ref[...] = jnp.where(better, tmax_b16, best_ref[...])
    bidx_ref[...] = jnp.where(better, targ, bidx_ref[...])

    @pl.when(k == _NK - 1)
    def _():
        idx_ref[...] = bidx_ref[...]


def _argmax_first_body(r_ref, emb_ref, idx_ref, best_ref, bidx_ref):
    @pl.when(pl.program_id(1) == 0)
    def _():
        best_ref[...] = jnp.full_like(best_ref, -jnp.inf)
        bidx_ref[...] = jnp.zeros_like(bidx_ref)

    _vq_tile(r_ref[...], emb_ref, idx_ref, best_ref, bidx_ref)


def _argmax_update_body(r_ref, q_ref, emb_ref, idx_ref, rout_ref,
                        best_ref, bidx_ref):
    @pl.when(pl.program_id(1) == 0)
    def _():
        rout_ref[...] = r_ref[...] - q_ref[...]
        best_ref[...] = jnp.full_like(best_ref, -jnp.inf)
        bidx_ref[...] = jnp.zeros_like(bidx_ref)

    _vq_tile(rout_ref[...], emb_ref, idx_ref, best_ref, bidx_ref)


def _argmax_first(r, emb):
    return pl.pallas_call(
        _argmax_first_body,
        grid=(_NR, _NK),
        in_specs=[
            pl.BlockSpec((_RT, _D), lambda ri, ki: (ri, 0)),
            pl.BlockSpec((_KT, _D), lambda ri, ki: (ki, 0)),
        ],
        out_specs=pl.BlockSpec((_RT, 1), lambda ri, ki: (ri, 0)),
        out_shape=jax.ShapeDtypeStruct((_M, 1), jnp.int32),
        scratch_shapes=[pltpu.VMEM((_RT, 1), jnp.float32),
                        pltpu.VMEM((_RT, 1), jnp.int32)],
        compiler_params=pltpu.CompilerParams(
            dimension_semantics=("parallel", "arbitrary")),
    )(r, emb)


def _argmax_update(r, q, emb):
    return pl.pallas_call(
        _argmax_update_body,
        grid=(_NR, _NK),
        in_specs=[
            pl.BlockSpec((_RT, _D), lambda ri, ki: (ri, 0)),
            pl.BlockSpec((_RT, _D), lambda ri, ki: (ri, 0)),
            pl.BlockSpec((_KT, _D), lambda ri, ki: (ki, 0)),
        ],
        out_specs=[
            pl.BlockSpec((_RT, 1), lambda ri, ki: (ri, 0)),
            pl.BlockSpec((_RT, _D), lambda ri, ki: (ri, 0)),
        ],
        out_shape=[jax.ShapeDtypeStruct((_M, 1), jnp.int32),
                   jax.ShapeDtypeStruct((_M, _D), jnp.float32)],
        scratch_shapes=[pltpu.VMEM((_RT, 1), jnp.float32),
                        pltpu.VMEM((_RT, 1), jnp.int32)],
        input_output_aliases={0: 1},
        compiler_params=pltpu.CompilerParams(
            dimension_semantics=("parallel", "arbitrary")),
    )(r, q, emb)


@functools.cache
def _sc_gather_kernel():
    mesh = plsc.VectorSubcoreMesh(core_axis_name="c", subcore_axis_name="s",
                                  num_cores=_NC, num_subcores=_NS)

    @functools.partial(
        pl.kernel, mesh=mesh,
        out_type=jax.ShapeDtypeStruct((_M, _D), jnp.float32),
        scratch_types=[pltpu.VMEM((_BPW,), jnp.int32),
                       pltpu.VMEM((_BPW, _D), jnp.float32),
                       pltpu.SemaphoreType.DMA],
    )
    def gather(table_hbm, idx_hbm, out_hbm, idx_v, rows_v, sem):
        wid = lax.axis_index("s") * _NC + lax.axis_index("c")
        base = wid * _BPW
        pltpu.sync_copy(idx_hbm.at[pl.ds(base, _BPW)], idx_v)
        pltpu.async_copy(table_hbm.at[idx_v], rows_v, sem).wait()
        pltpu.sync_copy(rows_v, out_hbm.at[pl.ds(base, _BPW)])

    return gather


def _gather(table, idx):
    return _sc_gather_kernel()(table, idx)


def _combine_body(x_ref, r_ref, q_ref, o_ref):
    o_ref[...] = x_ref[...] - r_ref[...] + q_ref[...]


def _combine(x, r, q):
    spec = pl.BlockSpec((_RT, _D), lambda i: (i, 0))
    return pl.pallas_call(
        _combine_body,
        grid=(_NR,),
        in_specs=[spec, spec, spec],
        out_specs=spec,
        out_shape=jax.ShapeDtypeStruct((_M, _D), jnp.float32),
    )(x, r, q)


def kernel(z, books):
    x = jnp.transpose(z, (0, 2, 1)).reshape(_M, _D)
    idx = _argmax_first(x, books[0])
    q = _gather(books[0], idx.reshape(_M))
    r = x
    for i in range(1, _NBOOKS):
        idx, r = _argmax_update(r, q, books[i])
        q = _gather(books[i], idx.reshape(_M))
    out = _combine(x, r, q)
    return jnp.transpose(out.reshape(_B, _T, _D), (0, 2, 1))
